# Initial kernel scaffold; baseline (speedup 1.0000x reference)
import jax
import jax.numpy as jnp
from jax.experimental import pallas as pl


def kernel(inputs, router_logits, w0, w1, wo):
    raise NotImplementedError("write your pallas kernel here")


# speedup vs baseline: 1.2114x; 1.2114x over previous
import functools

import jax
import jax.numpy as jnp
from jax import lax
from jax.experimental import pallas as pl
from jax.experimental.pallas import tpu as pltpu
from jax.experimental.pallas import tpu_sc as plsc

E = 8
TOP_K = 2
HIDDEN = 1024
INTER = 2048
TOKENS = 2048

TM = 128
CAP = TOKENS * TOP_K + E * TM
NT = CAP // TM
NMAP = 64

NW = 32
CHUNK = TOKENS // NW


def _routing_body(logits_ref, idx_ref, w_ref, map_ref):
    lg = logits_ref[...]
    lanes = lax.broadcasted_iota(jnp.int32, (TOKENS, E), 1)

    m1 = jnp.max(lg, axis=1, keepdims=True)
    e0 = jnp.min(jnp.where(lg == m1, lanes, E), axis=1, keepdims=True)
    lg2 = jnp.where(lanes == e0, -1e30, lg)
    m2 = jnp.max(lg2, axis=1, keepdims=True)
    e1 = jnp.min(jnp.where(lg2 == m2, lanes, E), axis=1, keepdims=True)

    wa = 1.0 / (1.0 + jnp.exp(m2 - m1))
    wb = 1.0 - wa

    oh0 = (lanes == e0).astype(jnp.float32)
    oh1 = (lanes == e1).astype(jnp.float32)
    oh = oh0 + oh1

    r = lax.broadcasted_iota(jnp.int32, (TOKENS, TOKENS), 0)
    c = lax.broadcasted_iota(jnp.int32, (TOKENS, TOKENS), 1)
    tri = (r > c).astype(jnp.float32)
    cc = jnp.dot(tri, oh, preferred_element_type=jnp.float32)

    counts = jnp.sum(oh, axis=0, keepdims=True)
    rc = jnp.ceil(counts / TM) * TM
    er = lax.broadcasted_iota(jnp.int32, (E, E), 0)
    ec = lax.broadcasted_iota(jnp.int32, (E, E), 1)
    excl = (er < ec).astype(jnp.float32)
    off = jnp.dot(rc, excl, preferred_element_type=jnp.float32)
    ends = off + rc

    pos = off + cc
    d0 = jnp.sum(oh0 * pos, axis=1, keepdims=True)
    d1 = jnp.sum(oh1 * pos, axis=1, keepdims=True)

    idx_ref[...] = jnp.where(
        lanes == 0, d0, jnp.where(lanes == 1, d1, 0.0)
    ).astype(jnp.int32)
    w_ref[...] = jnp.where(lanes == 0, wa, jnp.where(lanes == 1, wb, 0.0))

    tstart = lax.broadcasted_iota(jnp.int32, (NMAP, E), 0).astype(
        jnp.float32) * TM
    te = jnp.sum((tstart >= ends).astype(jnp.float32), axis=1, keepdims=True)
    te = jnp.minimum(te, E - 1)
    map_ref[...] = jnp.broadcast_to(te, (NMAP, E)).astype(jnp.int32)


def _routing(router_logits):
    return pl.pallas_call(
        _routing_body,
        out_shape=[
            jax.ShapeDtypeStruct((TOKENS, E), jnp.int32),
            jax.ShapeDtypeStruct((TOKENS, E), jnp.float32),
            jax.ShapeDtypeStruct((NMAP, E), jnp.int32),
        ],
    )(router_logits)


def _s1_body(x_hbm, d0_hbm, d1_hbm, xpad_hbm, d0_v, d1_v, rows_v, sem):
    wid = lax.axis_index("s") * 2 + lax.axis_index("c")
    base = wid * CHUNK
    pltpu.sync_copy(d0_hbm.at[pl.ds(base, CHUNK)], d0_v)
    pltpu.sync_copy(d1_hbm.at[pl.ds(base, CHUNK)], d1_v)
    pltpu.sync_copy(x_hbm.at[pl.ds(base, CHUNK)], rows_v)
    pltpu.async_copy(rows_v, xpad_hbm.at[d0_v], sem).wait()
    pltpu.async_copy(rows_v, xpad_hbm.at[d1_v], sem).wait()


def _scatter_inputs(x, d0, d1):
    mesh = plsc.VectorSubcoreMesh(core_axis_name="c", subcore_axis_name="s")
    return pl.kernel(
        _s1_body,
        out_type=jax.ShapeDtypeStruct((CAP, HIDDEN), jnp.float32),
        mesh=mesh,
        scratch_types=[
            pltpu.VMEM((CHUNK,), jnp.int32),
            pltpu.VMEM((CHUNK,), jnp.int32),
            pltpu.VMEM((CHUNK, HIDDEN), jnp.float32),
            pltpu.SemaphoreType.DMA,
        ],
    )(x, d0, d1)


def _up_body(map_ref, x_ref, w0_ref, w1_ref, h_ref):
    xb = x_ref[...].astype(jnp.bfloat16)
    a = jnp.dot(xb, w0_ref[0], preferred_element_type=jnp.float32)
    b = jnp.dot(xb, w1_ref[0], preferred_element_type=jnp.float32)
    h = (a * jax.nn.sigmoid(a)) * b
    h_ref[...] = h.astype(jnp.bfloat16)


def _up(x_pad, w0b, w1b, tile_map):
    grid_spec = pltpu.PrefetchScalarGridSpec(
        num_scalar_prefetch=1,
        grid=(NT,),
        in_specs=[
            pl.BlockSpec((TM, HIDDEN), lambda t, m: (t, 0)),
            pl.BlockSpec((1, HIDDEN, INTER), lambda t, m: (m[t], 0, 0)),
            pl.BlockSpec((1, HIDDEN, INTER), lambda t, m: (m[t], 0, 0)),
        ],
        out_specs=pl.BlockSpec((TM, INTER), lambda t, m: (t, 0)),
    )
    return pl.pallas_call(
        _up_body,
        grid_spec=grid_spec,
        out_shape=jax.ShapeDtypeStruct((CAP, INTER), jnp.bfloat16),
    )(tile_map, x_pad, w0b, w1b)


def _down_body(map_ref, h_ref, wo_ref, o_ref):
    o_ref[...] = jnp.dot(
        h_ref[...], wo_ref[0], preferred_element_type=jnp.float32)


def _down(hidden, wob, tile_map):
    grid_spec = pltpu.PrefetchScalarGridSpec(
        num_scalar_prefetch=1,
        grid=(NT,),
        in_specs=[
            pl.BlockSpec((TM, INTER), lambda t, m: (t, 0)),
            pl.BlockSpec((1, INTER, HIDDEN), lambda t, m: (m[t], 0, 0)),
        ],
        out_specs=pl.BlockSpec((TM, HIDDEN), lambda t, m: (t, 0)),
    )
    return pl.pallas_call(
        _down_body,
        grid_spec=grid_spec,
        out_shape=jax.ShapeDtypeStruct((CAP, HIDDEN), jnp.float32),
    )(tile_map, hidden, wob)


def _s2_body(os_hbm, d0_hbm, d1_hbm, g0_hbm, g1_hbm, d0_v, d1_v, rows_v, sem):
    wid = lax.axis_index("s") * 2 + lax.axis_index("c")
    base = wid * CHUNK
    pltpu.sync_copy(d0_hbm.at[pl.ds(base, CHUNK)], d0_v)
    pltpu.sync_copy(d1_hbm.at[pl.ds(base, CHUNK)], d1_v)
    pltpu.async_copy(os_hbm.at[d0_v], rows_v, sem).wait()
    pltpu.sync_copy(rows_v, g0_hbm.at[pl.ds(base, CHUNK)])
    pltpu.async_copy(os_hbm.at[d1_v], rows_v, sem).wait()
    pltpu.sync_copy(rows_v, g1_hbm.at[pl.ds(base, CHUNK)])


def _gather_outputs(out_sorted, d0, d1):
    mesh = plsc.VectorSubcoreMesh(core_axis_name="c", subcore_axis_name="s")
    return pl.kernel(
        _s2_body,
        out_type=[
            jax.ShapeDtypeStruct((TOKENS, HIDDEN), jnp.float32),
            jax.ShapeDtypeStruct((TOKENS, HIDDEN), jnp.float32),
        ],
        mesh=mesh,
        scratch_types=[
            pltpu.VMEM((CHUNK,), jnp.int32),
            pltpu.VMEM((CHUNK,), jnp.int32),
            pltpu.VMEM((CHUNK, HIDDEN), jnp.float32),
            pltpu.SemaphoreType.DMA,
        ],
    )(out_sorted, d0, d1)


def _combine_body(g0_ref, g1_ref, w_ref, o_ref):
    wa = w_ref[:, 0:1]
    wb = w_ref[:, 1:2]
    o_ref[...] = wa * g0_ref[...] + wb * g1_ref[...]


def _combine(g0, g1, w_out):
    nb = 4
    tb = TOKENS // nb
    return pl.pallas_call(
        _combine_body,
        grid=(nb,),
        in_specs=[
            pl.BlockSpec((tb, HIDDEN), lambda i: (i, 0)),
            pl.BlockSpec((tb, HIDDEN), lambda i: (i, 0)),
            pl.BlockSpec((tb, E), lambda i: (i, 0)),
        ],
        out_specs=pl.BlockSpec((tb, HIDDEN), lambda i: (i, 0)),
        out_shape=jax.ShapeDtypeStruct((TOKENS, HIDDEN), jnp.float32),
    )(g0, g1, w_out)


@jax.jit
def kernel(inputs, router_logits, w0, w1, wo):
    w0b = w0.astype(jnp.bfloat16)
    w1b = w1.astype(jnp.bfloat16)
    wob = wo.astype(jnp.bfloat16)

    idx_out, w_out, map_out = _routing(router_logits)
    d0 = idx_out[:, 0]
    d1 = idx_out[:, 1]
    tile_map = map_out[:NT, 0]

    x_pad = _scatter_inputs(inputs, d0, d1)
    hidden = _up(x_pad, w0b, w1b, tile_map)
    out_sorted = _down(hidden, wob, tile_map)
    g0, g1 = _gather_outputs(out_sorted, d0, d1)
    return _combine(g0, g1, w_out)


# speedup vs baseline: 1.3313x; 1.0990x over previous
import functools

import jax
import jax.numpy as jnp
from jax import lax
from jax.experimental import pallas as pl
from jax.experimental.pallas import tpu as pltpu
from jax.experimental.pallas import tpu_sc as plsc

E = 8
TOP_K = 2
HIDDEN = 1024
INTER = 2048
TOKENS = 2048

TM = 128
CAP = TOKENS * TOP_K + E * TM
NT = CAP // TM
NMAP = 64

NW = 32
CHUNK = TOKENS // NW


def _routing_body(logits_ref, idx_ref, w_ref, map_ref):
    lg = logits_ref[...]
    lanes = lax.broadcasted_iota(jnp.int32, (TOKENS, E), 1)

    m1 = jnp.max(lg, axis=1, keepdims=True)
    e0 = jnp.min(jnp.where(lg == m1, lanes, E), axis=1, keepdims=True)
    lg2 = jnp.where(lanes == e0, -1e30, lg)
    m2 = jnp.max(lg2, axis=1, keepdims=True)
    e1 = jnp.min(jnp.where(lg2 == m2, lanes, E), axis=1, keepdims=True)

    wa = 1.0 / (1.0 + jnp.exp(m2 - m1))
    wb = 1.0 - wa

    oh0 = (lanes == e0).astype(jnp.float32)
    oh1 = (lanes == e1).astype(jnp.float32)
    oh = oh0 + oh1

    r = lax.broadcasted_iota(jnp.int32, (TOKENS, TOKENS), 0)
    c = lax.broadcasted_iota(jnp.int32, (TOKENS, TOKENS), 1)
    tri = (r > c).astype(jnp.float32)
    cc = jnp.dot(tri, oh, preferred_element_type=jnp.float32)

    counts = jnp.sum(oh, axis=0, keepdims=True)
    rc = jnp.ceil(counts / TM) * TM
    er = lax.broadcasted_iota(jnp.int32, (E, E), 0)
    ec = lax.broadcasted_iota(jnp.int32, (E, E), 1)
    excl = (er < ec).astype(jnp.float32)
    off = jnp.dot(rc, excl, preferred_element_type=jnp.float32)
    ends = off + rc

    pos = off + cc
    d0 = jnp.sum(oh0 * pos, axis=1, keepdims=True)
    d1 = jnp.sum(oh1 * pos, axis=1, keepdims=True)

    idx_ref[...] = jnp.where(
        lanes == 0, d0, jnp.where(lanes == 1, d1, 0.0)
    ).astype(jnp.int32)
    w_ref[...] = jnp.where(lanes == 0, wa, jnp.where(lanes == 1, wb, 0.0))

    tstart = lax.broadcasted_iota(jnp.int32, (NMAP, E), 0).astype(
        jnp.float32) * TM
    te = jnp.sum((tstart >= ends).astype(jnp.float32), axis=1, keepdims=True)
    te = jnp.minimum(te, E - 1)
    map_ref[...] = jnp.broadcast_to(te, (NMAP, E)).astype(jnp.int32)


def _routing(router_logits):
    return pl.pallas_call(
        _routing_body,
        out_shape=[
            jax.ShapeDtypeStruct((TOKENS, E), jnp.int32),
            jax.ShapeDtypeStruct((TOKENS, E), jnp.float32),
            jax.ShapeDtypeStruct((NMAP, E), jnp.int32),
        ],
    )(router_logits)


def _s1_body(x_hbm, d0_hbm, d1_hbm, xpad_hbm, d0_v, d1_v, rows_v, semx, s0, s1):
    wid = lax.axis_index("s") * 2 + lax.axis_index("c")
    base = wid * CHUNK
    cx = pltpu.async_copy(x_hbm.at[pl.ds(base, CHUNK)], rows_v, semx)
    pltpu.sync_copy(d0_hbm.at[pl.ds(base, CHUNK)], d0_v)
    pltpu.sync_copy(d1_hbm.at[pl.ds(base, CHUNK)], d1_v)
    cx.wait()
    c0 = pltpu.async_copy(rows_v, xpad_hbm.at[d0_v], s0)
    c1 = pltpu.async_copy(rows_v, xpad_hbm.at[d1_v], s1)
    c0.wait()
    c1.wait()


def _scatter_inputs(x, d0, d1):
    mesh = plsc.VectorSubcoreMesh(core_axis_name="c", subcore_axis_name="s")
    return pl.kernel(
        _s1_body,
        out_type=jax.ShapeDtypeStruct((CAP, HIDDEN), jnp.float32),
        mesh=mesh,
        scratch_types=[
            pltpu.VMEM((CHUNK,), jnp.int32),
            pltpu.VMEM((CHUNK,), jnp.int32),
            pltpu.VMEM((CHUNK, HIDDEN), jnp.float32),
            pltpu.SemaphoreType.DMA,
            pltpu.SemaphoreType.DMA,
            pltpu.SemaphoreType.DMA,
        ],
    )(x, d0, d1)


def _ffn_body(map_ref, x_ref, w0_ref, w1_ref, wo_ref, o_ref):
    xb = x_ref[...].astype(jnp.bfloat16)
    a = jnp.dot(xb, w0_ref[0], preferred_element_type=jnp.float32)
    b = jnp.dot(xb, w1_ref[0], preferred_element_type=jnp.float32)
    h = ((a * jax.nn.sigmoid(a)) * b).astype(jnp.bfloat16)
    o_ref[...] = jnp.dot(h, wo_ref[0], preferred_element_type=jnp.float32)


def _ffn(x_pad, w0b, w1b, wob, tile_map):
    grid_spec = pltpu.PrefetchScalarGridSpec(
        num_scalar_prefetch=1,
        grid=(NT,),
        in_specs=[
            pl.BlockSpec((TM, HIDDEN), lambda t, m: (t, 0)),
            pl.BlockSpec((1, HIDDEN, INTER), lambda t, m: (m[t], 0, 0)),
            pl.BlockSpec((1, HIDDEN, INTER), lambda t, m: (m[t], 0, 0)),
            pl.BlockSpec((1, INTER, HIDDEN), lambda t, m: (m[t], 0, 0)),
        ],
        out_specs=pl.BlockSpec((TM, HIDDEN), lambda t, m: (t, 0)),
    )
    return pl.pallas_call(
        _ffn_body,
        grid_spec=grid_spec,
        out_shape=jax.ShapeDtypeStruct((CAP, HIDDEN), jnp.float32),
    )(tile_map, x_pad, w0b, w1b, wob)


HALF = CHUNK // 2


def _s2_body(os_hbm, d0_hbm, d1_hbm, g0_hbm, g1_hbm,
             d00_v, d01_v, d10_v, d11_v, r0_v, r1_v, s0, s1):
    wid = lax.axis_index("s") * 2 + lax.axis_index("c")
    base = wid * CHUNK
    pltpu.sync_copy(d0_hbm.at[pl.ds(base, HALF)], d00_v)
    pltpu.sync_copy(d0_hbm.at[pl.ds(base + HALF, HALF)], d01_v)
    pltpu.sync_copy(d1_hbm.at[pl.ds(base, HALF)], d10_v)
    pltpu.sync_copy(d1_hbm.at[pl.ds(base + HALF, HALF)], d11_v)
    for h, (da, db) in enumerate(((d00_v, d10_v), (d01_v, d11_v))):
        c0 = pltpu.async_copy(os_hbm.at[da], r0_v, s0)
        c1 = pltpu.async_copy(os_hbm.at[db], r1_v, s1)
        c0.wait()
        pltpu.sync_copy(r0_v, g0_hbm.at[pl.ds(base + h * HALF, HALF)])
        c1.wait()
        pltpu.sync_copy(r1_v, g1_hbm.at[pl.ds(base + h * HALF, HALF)])


def _gather_outputs(out_sorted, d0, d1):
    mesh = plsc.VectorSubcoreMesh(core_axis_name="c", subcore_axis_name="s")
    return pl.kernel(
        _s2_body,
        out_type=[
            jax.ShapeDtypeStruct((TOKENS, HIDDEN), jnp.float32),
            jax.ShapeDtypeStruct((TOKENS, HIDDEN), jnp.float32),
        ],
        mesh=mesh,
        scratch_types=[
            pltpu.VMEM((HALF,), jnp.int32),
            pltpu.VMEM((HALF,), jnp.int32),
            pltpu.VMEM((HALF,), jnp.int32),
            pltpu.VMEM((HALF,), jnp.int32),
            pltpu.VMEM((HALF, HIDDEN), jnp.float32),
            pltpu.VMEM((HALF, HIDDEN), jnp.float32),
            pltpu.SemaphoreType.DMA,
            pltpu.SemaphoreType.DMA,
        ],
    )(out_sorted, d0, d1)


def _combine_body(g0_ref, g1_ref, w_ref, o_ref):
    wa = w_ref[:, 0:1]
    wb = w_ref[:, 1:2]
    o_ref[...] = wa * g0_ref[...] + wb * g1_ref[...]


def _combine(g0, g1, w_out):
    nb = 4
    tb = TOKENS // nb
    return pl.pallas_call(
        _combine_body,
        grid=(nb,),
        in_specs=[
            pl.BlockSpec((tb, HIDDEN), lambda i: (i, 0)),
            pl.BlockSpec((tb, HIDDEN), lambda i: (i, 0)),
            pl.BlockSpec((tb, E), lambda i: (i, 0)),
        ],
        out_specs=pl.BlockSpec((tb, HIDDEN), lambda i: (i, 0)),
        out_shape=jax.ShapeDtypeStruct((TOKENS, HIDDEN), jnp.float32),
    )(g0, g1, w_out)


@jax.jit
def kernel(inputs, router_logits, w0, w1, wo):
    w0b = w0.astype(jnp.bfloat16)
    w1b = w1.astype(jnp.bfloat16)
    wob = wo.astype(jnp.bfloat16)

    idx_out, w_out, map_out = _routing(router_logits)
    d0 = idx_out[:, 0]
    d1 = idx_out[:, 1]
    tile_map = map_out[:NT, 0]

    x_pad = _scatter_inputs(inputs, d0, d1)
    out_sorted = _ffn(x_pad, w0b, w1b, wob, tile_map)
    g0, g1 = _gather_outputs(out_sorted, d0, d1)
    return _combine(g0, g1, w_out)


# speedup vs baseline: 1.8721x; 1.4062x over previous
import functools

import jax
import jax.numpy as jnp
from jax import lax
from jax.experimental import pallas as pl
from jax.experimental.pallas import tpu as pltpu
from jax.experimental.pallas import tpu_sc as plsc

E = 8
TOP_K = 2
HIDDEN = 1024
INTER = 2048
TOKENS = 2048

TM = 128
CAP = TOKENS * TOP_K + E * TM
NT = CAP // TM
NMAP = 64

NW = 32
CHUNK = TOKENS // NW


def _routing_body(lgt_ref, d_ref, w_ref, map_ref):
    lg = lgt_ref[...]
    sub = lax.broadcasted_iota(jnp.int32, (E, TOKENS), 0)

    m1 = jnp.max(lg, axis=0, keepdims=True)
    e0 = jnp.min(jnp.where(lg == m1, sub, E), axis=0, keepdims=True)
    lg2 = jnp.where(sub == e0, -1e30, lg)
    m2 = jnp.max(lg2, axis=0, keepdims=True)
    e1 = jnp.min(jnp.where(lg2 == m2, sub, E), axis=0, keepdims=True)

    wa = 1.0 / (1.0 + jnp.exp(m2 - m1))
    wb = 1.0 - wa

    oh0 = (sub == e0).astype(jnp.float32)
    oh1 = (sub == e1).astype(jnp.float32)
    oh = oh0 + oh1

    r = lax.broadcasted_iota(jnp.int32, (TOKENS, TOKENS), 0)
    c = lax.broadcasted_iota(jnp.int32, (TOKENS, TOKENS), 1)
    triu = (r < c).astype(jnp.float32)
    cc = jnp.dot(oh, triu, preferred_element_type=jnp.float32)

    counts = jnp.sum(oh, axis=1, keepdims=True)
    rc = jnp.ceil(counts / TM) * TM
    er = lax.broadcasted_iota(jnp.int32, (E, E), 0)
    ec = lax.broadcasted_iota(jnp.int32, (E, E), 1)
    excl = (ec < er).astype(jnp.float32)
    off = jnp.dot(excl, rc, preferred_element_type=jnp.float32)
    ends = off + rc

    pos = off + cc
    d0 = jnp.sum(oh0 * pos, axis=0, keepdims=True)
    d1 = jnp.sum(oh1 * pos, axis=0, keepdims=True)

    row2 = lax.broadcasted_iota(jnp.int32, (TOP_K, TOKENS), 0)
    d_ref[...] = jnp.where(row2 == 0, d0, d1).astype(jnp.int32)
    row3 = lax.broadcasted_iota(jnp.int32, (TOP_K, TOKENS, 16), 0)
    wa3 = lax.broadcast_in_dim(wa, (TOP_K, TOKENS, 16), (0, 1))
    wb3 = lax.broadcast_in_dim(wb, (TOP_K, TOKENS, 16), (0, 1))
    w_ref[...] = jnp.where(row3 == 0, wa3, wb3)

    tstart = lax.broadcasted_iota(jnp.int32, (E, NMAP), 1).astype(
        jnp.float32) * TM
    te = jnp.sum((tstart >= ends).astype(jnp.float32), axis=0, keepdims=True)
    te = jnp.minimum(te, E - 1)
    map_ref[...] = te.astype(jnp.int32)


def _routing(router_logits_t):
    return pl.pallas_call(
        _routing_body,
        out_shape=[
            jax.ShapeDtypeStruct((TOP_K, TOKENS), jnp.int32),
            jax.ShapeDtypeStruct((TOP_K, TOKENS, 16), jnp.float32),
            jax.ShapeDtypeStruct((1, NMAP), jnp.int32),
        ],
    )(router_logits_t)


def _s1_body(x_hbm, d_hbm, xpad_hbm, d0_v, d1_v, rows_v, semx, s0, s1):
    wid = lax.axis_index("s") * 2 + lax.axis_index("c")
    base = wid * CHUNK
    cx = pltpu.async_copy(x_hbm.at[pl.ds(base, CHUNK)], rows_v, semx)
    pltpu.sync_copy(d_hbm.at[0, pl.ds(base, CHUNK)], d0_v)
    pltpu.sync_copy(d_hbm.at[1, pl.ds(base, CHUNK)], d1_v)
    cx.wait()
    c0 = pltpu.async_copy(rows_v, xpad_hbm.at[d0_v], s0)
    c1 = pltpu.async_copy(rows_v, xpad_hbm.at[d1_v], s1)
    c0.wait()
    c1.wait()


def _scatter_inputs(x, d):
    mesh = plsc.VectorSubcoreMesh(core_axis_name="c", subcore_axis_name="s")
    return pl.kernel(
        _s1_body,
        out_type=jax.ShapeDtypeStruct((CAP, HIDDEN), jnp.float32),
        mesh=mesh,
        scratch_types=[
            pltpu.VMEM((CHUNK,), jnp.int32),
            pltpu.VMEM((CHUNK,), jnp.int32),
            pltpu.VMEM((CHUNK, HIDDEN), jnp.float32),
            pltpu.SemaphoreType.DMA,
            pltpu.SemaphoreType.DMA,
            pltpu.SemaphoreType.DMA,
        ],
    )(x, d)


def _ffn_body(map_ref, x_ref, w0_ref, w1_ref, wo_ref, o_ref):
    xb = x_ref[...].astype(jnp.bfloat16)
    w0b = w0_ref[0].astype(jnp.bfloat16)
    w1b = w1_ref[0].astype(jnp.bfloat16)
    a = jnp.dot(xb, w0b, preferred_element_type=jnp.float32)
    b = jnp.dot(xb, w1b, preferred_element_type=jnp.float32)
    h = ((a * jax.nn.sigmoid(a)) * b).astype(jnp.bfloat16)
    wob = wo_ref[0].astype(jnp.bfloat16)
    o_ref[...] = jnp.dot(h, wob, preferred_element_type=jnp.float32)


def _ffn(x_pad, w0, w1, wo, tile_map):
    grid_spec = pltpu.PrefetchScalarGridSpec(
        num_scalar_prefetch=1,
        grid=(NT,),
        in_specs=[
            pl.BlockSpec((TM, HIDDEN), lambda t, m: (t, 0)),
            pl.BlockSpec((1, HIDDEN, INTER), lambda t, m: (m[0, t], 0, 0)),
            pl.BlockSpec((1, HIDDEN, INTER), lambda t, m: (m[0, t], 0, 0)),
            pl.BlockSpec((1, INTER, HIDDEN), lambda t, m: (m[0, t], 0, 0)),
        ],
        out_specs=pl.BlockSpec((TM, HIDDEN), lambda t, m: (t, 0)),
    )
    return pl.pallas_call(
        _ffn_body,
        grid_spec=grid_spec,
        out_shape=jax.ShapeDtypeStruct((CAP, HIDDEN), jnp.float32),
    )(tile_map, x_pad, w0, w1, wo)


HALF = CHUNK // 2


def _s2_body(os_hbm, d_hbm, wt_hbm, out_hbm,
             d00_v, d01_v, d10_v, d11_v, wa_v, wb_v, r0_v, r1_v, s0, s1):
    wid = lax.axis_index("s") * 2 + lax.axis_index("c")
    base = wid * CHUNK
    pltpu.sync_copy(wt_hbm.at[0, pl.ds(base, CHUNK)], wa_v)
    pltpu.sync_copy(wt_hbm.at[1, pl.ds(base, CHUNK)], wb_v)
    pltpu.sync_copy(d_hbm.at[0, pl.ds(base, HALF)], d00_v)
    pltpu.sync_copy(d_hbm.at[0, pl.ds(base + HALF, HALF)], d01_v)
    pltpu.sync_copy(d_hbm.at[1, pl.ds(base, HALF)], d10_v)
    pltpu.sync_copy(d_hbm.at[1, pl.ds(base + HALF, HALF)], d11_v)
    for h, (da, db) in enumerate(((d00_v, d10_v), (d01_v, d11_v))):
        c0 = pltpu.async_copy(os_hbm.at[da], r0_v, s0)
        c1 = pltpu.async_copy(os_hbm.at[db], r1_v, s1)
        c0.wait()
        c1.wait()

        def tok_body(j, _):
            tok = h * HALF + j
            wsa = wa_v[tok]
            wsb = wb_v[tok]

            def col_body(ci, _):
                sl = pl.ds(ci * 16, 16)
                r0_v[j, sl] = wsa * r0_v[j, sl] + wsb * r1_v[j, sl]
                return 0

            lax.fori_loop(0, HIDDEN // 16, col_body, 0, unroll=4)
            return 0

        lax.fori_loop(0, HALF, tok_body, 0)
        pltpu.sync_copy(r0_v, out_hbm.at[pl.ds(base + h * HALF, HALF)])


def _gather_combine(out_sorted, d, wt):
    mesh = plsc.VectorSubcoreMesh(core_axis_name="c", subcore_axis_name="s")
    return pl.kernel(
        _s2_body,
        out_type=jax.ShapeDtypeStruct((TOKENS, HIDDEN), jnp.float32),
        mesh=mesh,
        scratch_types=[
            pltpu.VMEM((HALF,), jnp.int32),
            pltpu.VMEM((HALF,), jnp.int32),
            pltpu.VMEM((HALF,), jnp.int32),
            pltpu.VMEM((HALF,), jnp.int32),
            pltpu.VMEM((CHUNK, 16), jnp.float32),
            pltpu.VMEM((CHUNK, 16), jnp.float32),
            pltpu.VMEM((HALF, HIDDEN), jnp.float32),
            pltpu.VMEM((HALF, HIDDEN), jnp.float32),
            pltpu.SemaphoreType.DMA,
            pltpu.SemaphoreType.DMA,
        ],
    )(out_sorted, d, wt)


@jax.jit
def kernel(inputs, router_logits, w0, w1, wo):
    d, wt, tile_map = _routing(router_logits.T)
    x_pad = _scatter_inputs(inputs, d)
    out_sorted = _ffn(x_pad, w0, w1, wo, tile_map)
    return _gather_combine(out_sorted, d, wt)
